# baseline (speedup 1.0000x reference)
import functools

import jax
import jax.numpy as jnp
from jax import lax
from jax.experimental import pallas as pl
from jax.experimental.pallas import tpu as pltpu
from jax.experimental.pallas import tpu_sc as plsc

B = 16384
F = 26
CARD = 100000
D = 32
FD = F * D
EPS = 1e-5

NC = 2
NS = 16
NW = NC * NS
BF = B * F
PER_W = BF // NW
CHUNK = 1024
N_CHUNKS = PER_W // CHUNK


def _sc_gather(flat_tab, idx):
    mesh = plsc.VectorSubcoreMesh(core_axis_name="c", subcore_axis_name="s")

    @functools.partial(
        pl.kernel,
        mesh=mesh,
        compiler_params=pltpu.CompilerParams(use_tc_tiling_on_sc=False),
        out_type=jax.ShapeDtypeStruct((BF, D), jnp.float32),
        scratch_types=[
            pltpu.VMEM((CHUNK,), jnp.int32),
            pltpu.VMEM((CHUNK, D), jnp.float32),
            pltpu.SemaphoreType.DMA,
        ],
    )
    def k(tab_hbm, idx_hbm, out_hbm, idx_v, rows_v, sem):
        wid = lax.axis_index("s") * NC + lax.axis_index("c")
        base0 = wid * PER_W
        for j in range(N_CHUNKS):
            base = base0 + j * CHUNK
            pltpu.sync_copy(idx_hbm.at[pl.ds(base, CHUNK)], idx_v)
            pltpu.async_copy(tab_hbm.at[idx_v], rows_v, sem).wait()
            pltpu.sync_copy(rows_v, out_hbm.at[pl.ds(base, CHUNK)])

    return k(flat_tab, idx)


BT = 1024


def _tc_body(emb_ref, w0, w1, w2, w3, b_ref, g_ref, bt_ref, s_ref, e_ref,
             out_ref):
    hi = jax.lax.Precision.HIGHEST
    e = emb_ref[...]
    c0 = jnp.dot(e[:, 0:256], w0[...], precision=hi)
    c1 = jnp.dot(e[:, 256:512], w1[...], precision=hi)
    c2 = jnp.dot(e[:, 512:768], w2[...], precision=hi)
    c3 = jnp.dot(e[:, 768:832], w3[...], precision=hi)
    c = jnp.concatenate([c0, c1, c2, c3], axis=1) + b_ref[...]
    sq = c * c
    msq = jnp.dot(sq, s_ref[...], precision=hi)
    r = lax.rsqrt(msq + EPS)
    scale = jnp.dot(r, e_ref[...], precision=hi)
    out_ref[...] = c * scale * g_ref[...] + bt_ref[...]


def _tc_norm(emb2, w0, w1, w2, w3, b832, g832, bt832, S, E):
    grid = (B // BT,)
    full = lambda shape: pl.BlockSpec(shape, lambda i: (0, 0))
    return pl.pallas_call(
        _tc_body,
        grid=grid,
        in_specs=[
            pl.BlockSpec((BT, FD), lambda i: (i, 0)),
            full((256, 256)), full((256, 256)), full((256, 256)),
            full((64, 64)),
            full((1, FD)), full((1, FD)), full((1, FD)),
            full((FD, 128)), full((128, FD)),
        ],
        out_specs=pl.BlockSpec((BT, FD), lambda i: (i, 0)),
        out_shape=jax.ShapeDtypeStruct((B, FD), jnp.float32),
    )(emb2, w0, w1, w2, w3, b832, g832, bt832, S, E)


def kernel(x, tables, proj_W, proj_b, gamma, beta):
    offs = (jnp.arange(F, dtype=jnp.int32) * (CARD + 1))[None, :]
    idx = (jnp.clip(x, 0, CARD).astype(jnp.int32) + offs).reshape(-1)
    flat_tab = tables.reshape(F * (CARD + 1), D)

    C = jnp.eye(D, dtype=jnp.float32) - jnp.full((D, D), 1.0 / D,
                                                 dtype=jnp.float32)
    Wc = jnp.matmul(proj_W, C)
    bc = jnp.matmul(proj_b, C)

    blkdiag = jax.scipy.linalg.block_diag
    w0 = blkdiag(*[Wc[f] for f in range(0, 8)])
    w1 = blkdiag(*[Wc[f] for f in range(8, 16)])
    w2 = blkdiag(*[Wc[f] for f in range(16, 24)])
    w3 = blkdiag(*[Wc[f] for f in range(24, 26)])
    b832 = bc.reshape(1, FD)
    g832 = jnp.tile(gamma, F)[None, :]
    bt832 = jnp.tile(beta, F)[None, :]

    d_ids = jnp.arange(FD, dtype=jnp.int32) // D
    S = (d_ids[:, None] == jnp.arange(128, dtype=jnp.int32)[None, :]
         ).astype(jnp.float32) / D
    E = (jnp.arange(128, dtype=jnp.int32)[:, None] == d_ids[None, :]
         ).astype(jnp.float32)

    emb = _sc_gather(flat_tab, idx)
    emb2 = emb.reshape(B, FD)
    out2 = _tc_norm(emb2, w0, w1, w2, w3, b832, g832, bt832, S, E)
    return out2.reshape(B, F, D)
